# Initial kernel scaffold; baseline (speedup 1.0000x reference)
import jax
import jax.numpy as jnp
from jax.experimental import pallas as pl


def kernel(feature, jmap, lines, W1, b1, W2a, b2a, W2b, b2b, W2c, b2c):
    raise NotImplementedError("write your pallas kernel here")


# speedup vs baseline: 5.1474x; 5.1474x over previous
import functools

import jax
import jax.numpy as jnp
from jax import lax
from jax.experimental import pallas as pl
from jax.experimental.pallas import tpu as pltpu
from jax.experimental.pallas import tpu_sc as plsc

N_PTS0 = 32
N_PTS1 = 8
DIM_LOI = 128
HH = 128
WW = 128
HW = HH * WW

NL_PAD = 10240
M_PAD = NL_PAD * N_PTS0
G_PAD = M_PAD // 4
LB = 128

NWORK = 32
PTS_PER_TILE = M_PAD // NWORK
SUP = 1280
NSUP = PTS_PER_TILE // SUP
CH = 64
NCH = SUP // CH
GCH = CH // 4


def _fc1_body(f_ref, w_ref, b_ref, o_ref):
    f = f_ref[0]
    w = w_ref[...]
    acc = lax.dot_general(f, w, (((0,), (1,)), ((), ())),
                          preferred_element_type=jnp.float32)
    o_ref[0] = acc + b_ref[...]


def _fc1(feature, W1, b1):
    Bb = feature.shape[0]
    HWB = 2048
    grid = (Bb, HW // HWB)
    return pl.pallas_call(
        _fc1_body,
        grid=grid,
        in_specs=[
            pl.BlockSpec((1, 256, HWB), lambda b, h: (b, 0, h)),
            pl.BlockSpec((DIM_LOI, 256), lambda b, h: (0, 0)),
            pl.BlockSpec((1, DIM_LOI), lambda b, h: (0, 0)),
        ],
        out_specs=pl.BlockSpec((1, HWB, DIM_LOI), lambda b, h: (b, h, 0)),
        out_shape=jax.ShapeDtypeStruct((Bb, HW, DIM_LOI), jnp.float32),
    )(feature.reshape(Bb, 256, HW), W1, b1.reshape(1, DIM_LOI))


def _pts_body(l_ref, lam_ref, *out_refs):
    g = pl.program_id(0)
    r = l_ref[...]
    x0 = r[0:1, :]
    y0 = r[1:2, :]
    x1 = r[2:3, :]
    y1 = r[3:4, :]
    lam = lam_ref[...]
    oml = 1.0 - lam
    px = x0 * lam + x1 * oml - 0.5
    py = y0 * lam + y1 * oml - 0.5
    px0 = jnp.clip(jnp.floor(px), 0.0, HH - 1.0)
    py0 = jnp.clip(jnp.floor(py), 0.0, WW - 1.0)
    px1 = jnp.clip(px0 + 1.0, 0.0, HH - 1.0)
    py1 = jnp.clip(py0 + 1.0, 0.0, WW - 1.0)
    px0l = px0.astype(jnp.int32)
    py0l = py0.astype(jnp.int32)
    px1l = px1.astype(jnp.int32)
    py1l = py1.astype(jnp.int32)
    cols = g * LB + lax.broadcasted_iota(jnp.int32, (N_PTS0, LB), 1)
    boff = (jnp.where(cols >= 5000, 16384, 0)
            + jnp.where(cols >= 10000, 16384, 0))
    idx = [
        jnp.minimum(boff + px0l * WW + py0l, 2 * HW - 1),
        jnp.minimum(boff + px1l * WW + py0l, 2 * HW - 1),
        jnp.minimum(boff + px0l * WW + py1l, 2 * HW - 1),
        jnp.minimum(boff + px1l * WW + py1l, 2 * HW - 1),
    ]
    ws = [
        (px1 - px) * (py1 - py),
        (px - px0) * (py1 - py),
        (px1 - px) * (py - py0),
        (px - px0) * (py - py0),
    ]
    for c in range(4):
        out_refs[c][...] = idx[c].T
        out_refs[4 + c][...] = lax.broadcast_in_dim(
            ws[c].T, (LB, N_PTS0, 16), (0, 1))


def _points(lines_tp):
    grid = (NL_PAD // LB,)
    obs_i = pl.BlockSpec((LB, N_PTS0), lambda g: (g, 0))
    obs_w = pl.BlockSpec((LB, N_PTS0, 16), lambda g: (g, 0, 0))
    oshape_i = jax.ShapeDtypeStruct((NL_PAD, N_PTS0), jnp.int32)
    oshape_w = jax.ShapeDtypeStruct((NL_PAD, N_PTS0, 16), jnp.float32)
    outs = pl.pallas_call(
        _pts_body,
        grid=grid,
        in_specs=[pl.BlockSpec((4, LB), lambda g: (0, g)),
                  pl.BlockSpec((N_PTS0, 1), lambda g: (0, 0))],
        out_specs=[obs_i] * 4 + [obs_w] * 4,
        out_shape=[oshape_i] * 4 + [oshape_w] * 4,
    )(lines_tp, jnp.linspace(0.0, 1.0, N_PTS0).reshape(N_PTS0, 1))
    return ([o.reshape(M_PAD) for o in outs[:4]],
            [o.reshape(M_PAD, 16) for o in outs[4:]])


def _sc_body(xt, i0, i1, i2, i3, w0, w1, w2, w3, out,
             is0, is1, is2, is3, ws0, ws1, ws2, ws3,
             r0, r1, r2, r3, ov, sem):
    wid = lax.axis_index("s") * 2 + lax.axis_index("c")
    pbase0 = wid * PTS_PER_TILE
    idx_h = [i0, i1, i2, i3]
    w_h = [w0, w1, w2, w3]
    idx_s = [is0, is1, is2, is3]
    w_s = [ws0, ws1, ws2, ws3]
    rows = [r0, r1, r2, r3]

    def sup_body(s, _):
        pbase = pl.multiple_of(pbase0 + s * SUP, SUP)
        for c in range(4):
            pltpu.sync_copy(idx_h[c].at[pl.ds(pbase, SUP)], idx_s[c])

        def ch_body(ci, _):
            poff = pl.multiple_of(pbase + ci * CH, CH)
            cps = [pltpu.async_copy(
                       xt.at[idx_s[c].at[pl.ds(ci * CH, CH)]], rows[c], sem)
                   for c in range(4)]
            for c in range(4):
                pltpu.sync_copy(w_h[c].at[pl.ds(poff, CH)], w_s[c])
            for cp in cps:
                cp.wait()

            def g_body(gi, _):
                accs = [None] * 8
                for j in range(4):
                    p = gi * 4 + j
                    wb = [w_s[c][p, :] for c in range(4)]
                    for k in range(8):
                        sl = pl.ds(k * 16, 16)
                        v = (rows[0][p, sl] * wb[0]
                             + rows[1][p, sl] * wb[1]
                             + rows[2][p, sl] * wb[2]
                             + rows[3][p, sl] * wb[3])
                        accs[k] = v if j == 0 else jnp.maximum(accs[k], v)
                for k in range(8):
                    ov[gi, pl.ds(k * 16, 16)] = accs[k]
                return 0

            lax.fori_loop(0, GCH, g_body, 0)
            gb = pl.multiple_of((pbase + ci * CH) // 4, GCH)
            pltpu.sync_copy(ov, out.at[pl.ds(gb, GCH)])
            return 0

        lax.fori_loop(0, NCH, ch_body, 0)
        return 0

    lax.fori_loop(0, NSUP, sup_body, 0)


def _sc_gather(xt, idxs, ws):
    mesh = plsc.VectorSubcoreMesh(core_axis_name="c", subcore_axis_name="s")
    fn = pl.kernel(
        _sc_body,
        out_type=jax.ShapeDtypeStruct((G_PAD, DIM_LOI), jnp.float32),
        mesh=mesh,
        scratch_types=(
            [pltpu.VMEM((SUP,), jnp.int32) for _ in range(4)]
            + [pltpu.VMEM((CH, 16), jnp.float32) for _ in range(4)]
            + [pltpu.VMEM((CH, DIM_LOI), jnp.float32) for _ in range(4)]
            + [pltpu.VMEM((GCH, DIM_LOI), jnp.float32),
               pltpu.SemaphoreType.DMA]),
    )
    return fn(xt, *idxs, *ws)


def _mlp_body(f_ref, wa_ref, ba_ref, wb_ref, bb_ref, wc_ref, bc_ref,
              lo_ref, bo_ref):
    f = f_ref[...]
    h1 = lax.dot_general(f, wa_ref[...], (((1,), (1,)), ((), ())),
                         preferred_element_type=jnp.float32) + ba_ref[...]
    h1 = jnp.maximum(h1, 0.0)
    h2 = lax.dot_general(h1, wb_ref[...], (((1,), (1,)), ((), ())),
                         preferred_element_type=jnp.float32) + bb_ref[...]
    h2 = jnp.maximum(h2, 0.0)
    logits = lax.dot_general(h2, wc_ref[...], (((1,), (1,)), ((), ())),
                             preferred_element_type=jnp.float32) + bc_ref[...]
    m = jnp.max(logits, axis=1, keepdims=True)
    e = jnp.exp(logits - m)
    s = e / jnp.sum(e, axis=1, keepdims=True)
    cond = ((s[:, 1:2] > 0.25) | (s[:, 2:3] > 0.25) | (s[:, 3:4] > 0.25)) \
        & (s[:, 0:1] < 0.25)
    lo_ref[...] = logits
    bo_ref[...] = jnp.where(cond, 1.0, 0.0)


def _mlp(feat, W2aP, b2a, W2b, b2b, W2c, b2c):
    R = 1000
    grid = (10000 // R,)
    return pl.pallas_call(
        _mlp_body,
        grid=grid,
        in_specs=[
            pl.BlockSpec((R, 1024), lambda g: (g, 0)),
            pl.BlockSpec((1024, 1024), lambda g: (0, 0)),
            pl.BlockSpec((1, 1024), lambda g: (0, 0)),
            pl.BlockSpec((1024, 1024), lambda g: (0, 0)),
            pl.BlockSpec((1, 1024), lambda g: (0, 0)),
            pl.BlockSpec((4, 1024), lambda g: (0, 0)),
            pl.BlockSpec((1, 4), lambda g: (0, 0)),
        ],
        out_specs=[
            pl.BlockSpec((R, 4), lambda g: (g, 0)),
            pl.BlockSpec((R, 1), lambda g: (g, 0)),
        ],
        out_shape=[
            jax.ShapeDtypeStruct((10000, 4), jnp.float32),
            jax.ShapeDtypeStruct((10000, 1), jnp.float32),
        ],
    )(feat, W2aP, b2a.reshape(1, 1024), W2b, b2b.reshape(1, 1024),
      W2c, b2c.reshape(1, 4))


def _nms_body(j_ref, o_ref):
    jm = j_ref[0]
    jm0 = jm[0]
    jm1 = jm[1]
    m = jnp.maximum(jm0, jm1)
    ninf_r = jnp.full((2, WW), -jnp.inf, jnp.float32)
    mp = jnp.concatenate([ninf_r, m, ninf_r], axis=0)
    vm = mp[0:HH, :]
    for i in range(1, 5):
        vm = jnp.maximum(vm, mp[i:i + HH, :])
    ninf_c = jnp.full((HH, 2), -jnp.inf, jnp.float32)
    hp = jnp.concatenate([ninf_c, vm, ninf_c], axis=1)
    hm = hp[:, 0:WW]
    for i in range(1, 5):
        hm = jnp.maximum(hm, hp[:, i:i + WW])
    o_ref[0, 0] = jnp.where(jm0 == hm, jm0, 0.0)
    o_ref[0, 1] = jnp.where(jm1 == hm, jm1, 0.0)


def _nms(jmap):
    Bb = jmap.shape[0]
    return pl.pallas_call(
        _nms_body,
        grid=(Bb,),
        in_specs=[pl.BlockSpec((1, 2, HH, WW), lambda b: (b, 0, 0, 0))],
        out_specs=pl.BlockSpec((1, 2, HH, WW), lambda b: (b, 0, 0, 0)),
        out_shape=jax.ShapeDtypeStruct((Bb, 2, HH, WW), jnp.float32),
    )(jmap)


def kernel(feature, jmap, lines, W1, b1, W2a, b2a, W2b, b2b, W2c, b2c):
    Bb = feature.shape[0]
    NLl = lines.shape[1]

    xt = _fc1(feature, W1, b1).reshape(Bb * HW, DIM_LOI)

    lt = lines.reshape(Bb * NLl, 4).T
    lines_tp = jnp.pad(lt, ((0, 0), (0, NL_PAD - Bb * NLl)))
    idxs, ws = _points(lines_tp)

    pooled = _sc_gather(xt, idxs, ws)

    feat = pooled[:Bb * NLl * N_PTS1].reshape(Bb * NLl, N_PTS1 * DIM_LOI)
    W2aP = W2a.reshape(1024, DIM_LOI, N_PTS1).transpose(0, 2, 1).reshape(1024, 1024)
    logits, bflag = _mlp(feat, W2aP, b2a, W2b, b2b, W2c, b2c)

    jmap_nms = _nms(jmap).reshape(Bb, 2, HW)

    b = bflag.reshape(Bb * NLl).astype(jnp.bool_)
    return logits, jmap_nms, b
